# baseline (speedup 1.0000x reference)
import jax
import jax.numpy as jnp
from jax import lax
from jax.experimental import pallas as pl
from jax.experimental.pallas import tpu as pltpu
from jax.experimental.pallas import tpu_sc as plsc


_NC, _NS, _LANES = 2, 16, 16
_NW = _NC * _NS
_K = 512
_FR = 904
_SC_N = 98304
_R = 8


def _tc_body(x_ref, cp_ref, nc_ref, hyd_ref, mol_ref, a0_ref, a1_ref, a2_ref,
             m_ref, w_ref, out_ref):
    R, L = x_ref.shape
    w = w_ref[...]
    iota = lax.broadcasted_iota(jnp.int32, (43, L), 0)
    for r in range(R):
        xi = x_ref[r:r + 1, :]
        cpi = cp_ref[r:r + 1, :]
        nci = nc_ref[r:r + 1, :]
        oh = ((iota == xi) | (iota == cpi + 32) | (iota == nci + 39))
        m = (m_ref[r:r + 1, :] != 0).astype(jnp.float32)
        notm = 1.0 - m
        ohf = oh.astype(jnp.float32) * notm

        def angrow(a_ref):
            a = a_ref[r:r + 1, :] / 180.0
            return jnp.where(jnp.isinf(a), 0.0, a) * notm

        a_mat = jnp.concatenate(
            [ohf,
             hyd_ref[r:r + 1, :] * notm,
             mol_ref[r:r + 1, :] * notm,
             angrow(a0_ref), angrow(a1_ref), angrow(a2_ref),
             jnp.broadcast_to(m, (9, L)),
             jnp.zeros((7, L), jnp.float32)], axis=0)

        h = lax.dot_general(a_mat, w, (((0,), (0,)), ((), ())),
                            preferred_element_type=jnp.float32)
        out_ref[pl.ds(r * L, L), :] = h


def _ftable_body(w_ref, am_ref, f_ref):
    c = lax.broadcasted_iota(jnp.int32, (_FR, 48), 0)
    k = lax.broadcasted_iota(jnp.int32, (_FR, 48), 1)
    oh = ((k == c // 28) | (k == 32 + (c % 28) // 4) | (k == 39 + c % 4))
    oh = oh & (c < 896)
    h = jnp.dot(oh.astype(jnp.float32), w_ref[...],
                preferred_element_type=jnp.float32)
    mr = jnp.sum(am_ref[...], axis=0, keepdims=True)
    ci = lax.broadcasted_iota(jnp.int32, (_FR, 1), 0)
    f_ref[...] = h + jnp.where(ci == 896, 1.0, 0.0) * mr


def _build_ftable(w48, atom_mask_embed):
    H = w48.shape[1]
    return pl.pallas_call(
        _ftable_body,
        in_specs=[pl.BlockSpec((48, H), lambda: (0, 0)),
                  pl.BlockSpec((9, H), lambda: (0, 0))],
        out_specs=pl.BlockSpec((_FR, H), lambda: (0, 0)),
        out_shape=jax.ShapeDtypeStruct((_FR, H), jnp.float32),
    )(w48, atom_mask_embed)


def _sc_body(f_hbm, x_hbm, cp_hbm, nc_hbm, hyd_hbm, mol_hbm,
             a0_hbm, a1_hbm, a2_hbm, m_hbm, wsm_hbm, out_hbm,
             xv, cpv, ncv, mv, hydv, molv, a0v, a1v, a2v,
             idx_v, rows_v, wsm_v, f_sh, sem):
    H = rows_v.shape[1]
    n = x_hbm.shape[0]
    sc_base = n - _SC_N
    per_w = _SC_N // _NW
    nch = per_w // _K
    sid = lax.axis_index("s")
    wid = sid * _NC + lax.axis_index("c")
    pltpu.sync_copy(wsm_hbm, wsm_v)

    @pl.when(sid == 0)
    def _stage():
        pltpu.sync_copy(f_hbm, f_sh)
    plsc.subcore_barrier()

    @pl.loop(0, nch)
    def _chunk(ch):
        base = sc_base + wid * per_w + ch * _K
        obase = wid * per_w + ch * _K
        pltpu.sync_copy(x_hbm.at[pl.ds(base, _K)], xv)
        pltpu.sync_copy(cp_hbm.at[pl.ds(base, _K)], cpv)
        pltpu.sync_copy(nc_hbm.at[pl.ds(base, _K)], ncv)
        pltpu.sync_copy(m_hbm.at[pl.ds(base, _K)], mv)
        pltpu.sync_copy(hyd_hbm.at[pl.ds(base, _K)], hydv)
        pltpu.sync_copy(mol_hbm.at[pl.ds(base, _K)], molv)
        pltpu.sync_copy(a0_hbm.at[pl.ds(base, _K)], a0v)
        pltpu.sync_copy(a1_hbm.at[pl.ds(base, _K)], a1v)
        pltpu.sync_copy(a2_hbm.at[pl.ds(base, _K)], a2v)

        @pl.loop(0, _K // _LANES)
        def _grp(g):
            sl = pl.ds(g * _LANES, _LANES)
            m16 = mv[sl]
            cidx = xv[sl] * 28 + cpv[sl] * 4 + ncv[sl]
            idx_v[sl] = jnp.where(m16 != 0,
                                  jnp.full((_LANES,), 896, jnp.int32), cidx)
            zf = jnp.zeros((_LANES,), jnp.float32)
            of = jnp.full((_LANES,), 1.0, jnp.float32)
            notm = jnp.where(m16 == 0, of, zf)
            hydv[sl] = hydv[sl] * notm
            molv[sl] = molv[sl] * notm
            inf = jnp.full((_LANES,), jnp.inf, jnp.float32)
            for av in (a0v, a1v, a2v):
                a = av[sl] / 180.0
                a = jnp.where(jnp.abs(a) == inf, zf, a)
                av[sl] = a * notm

        pltpu.async_copy(f_sh.at[idx_v], rows_v, sem).wait()

        @pl.loop(0, _K // _LANES)
        def _grp2(g):
            sl = pl.ds(g * _LANES, _LANES)
            svec = [hydv[sl], molv[sl], a0v[sl], a1v[sl], a2v[sl]]
            wq = [[wsm_v[j, pl.ds(q * _LANES, _LANES)] for q in range(8)]
                  for j in range(5)]
            dn = lax.GatherDimensionNumbers(
                offset_dims=(), collapsed_slice_dims=(0,),
                start_index_map=(0,))

            @pl.loop(0, _LANES)
            def _tok(ti):
                t = g * _LANES + ti
                idxv = jnp.full((_LANES,), ti, jnp.int32)
                sp = [lax.gather(s, idxv[:, None], dn, (1,),
                                 mode=lax.GatherScatterMode.PROMISE_IN_BOUNDS)
                      for s in svec]
                for q in range(8):
                    sl2 = pl.ds(q * _LANES, _LANES)
                    rows_v[t, sl2] = (rows_v[t, sl2] + sp[0] * wq[0][q] +
                                      sp[1] * wq[1][q] + sp[2] * wq[2][q] +
                                      sp[3] * wq[3][q] + sp[4] * wq[4][q])

        pltpu.sync_copy(rows_v, out_hbm.at[pl.ds(obase, _K), :])


def kernel(x, chem_polar, net_charge, hydropathy, mol_mass, ang, mask_aa,
           token_embed, atom_mask_embed, chem_polar_embed, net_charge_embed,
           hydropathy_W, mol_mass_W, angle_W):
    B, L = x.shape
    H = token_embed.shape[1]
    N = B * L
    n_tc = N - _SC_N
    b_tc = n_tc // L
    G = b_tc // _R

    w48 = jnp.concatenate(
        [token_embed, chem_polar_embed, net_charge_embed,
         jnp.zeros((5, H), jnp.float32)], axis=0)
    w64 = jnp.concatenate(
        [token_embed, chem_polar_embed, net_charge_embed,
         hydropathy_W.T, mol_mass_W.T, angle_W.T, atom_mask_embed,
         jnp.zeros((7, H), jnp.float32)], axis=0)
    wsm = jnp.concatenate(
        [hydropathy_W.T, mol_mass_W.T, angle_W.T,
         jnp.zeros((3, H), jnp.float32)], axis=0)
    f = _build_ftable(w48, atom_mask_embed)

    xi = x.astype(jnp.int32)
    cpi = chem_polar.astype(jnp.int32)
    nci = net_charge.astype(jnp.int32)
    hyd2 = hydropathy[..., 0]
    mol2 = mol_mass[..., 0]
    a02 = ang[..., 0]
    a12 = ang[..., 1]
    a22 = ang[..., 2]
    m2 = mask_aa[..., 0]

    tok = pl.BlockSpec((_R, L), lambda i: (i, 0))
    out_tc = pl.pallas_call(
        _tc_body,
        grid=(G,),
        in_specs=[tok] * 9 + [pl.BlockSpec((64, H), lambda i: (0, 0))],
        out_specs=pl.BlockSpec((_R * L, H), lambda i: (i, 0)),
        out_shape=jax.ShapeDtypeStruct((n_tc, H), jnp.float32),
        compiler_params=pltpu.CompilerParams(
            dimension_semantics=("arbitrary",)),
    )(xi, cpi, nci, hyd2, mol2, a02, a12, a22, m2, w64)

    flat = lambda v: v.reshape(N)
    sc = pl.kernel(
        _sc_body,
        out_type=jax.ShapeDtypeStruct((_SC_N, H), jnp.float32),
        mesh=plsc.VectorSubcoreMesh(core_axis_name="c", subcore_axis_name="s",
                                    num_cores=_NC, num_subcores=_NS),
        scratch_types=[
            pltpu.VMEM((_K,), jnp.int32),
            pltpu.VMEM((_K,), jnp.int32),
            pltpu.VMEM((_K,), jnp.int32),
            pltpu.VMEM((_K,), jnp.int32),
            pltpu.VMEM((_K,), jnp.float32),
            pltpu.VMEM((_K,), jnp.float32),
            pltpu.VMEM((_K,), jnp.float32),
            pltpu.VMEM((_K,), jnp.float32),
            pltpu.VMEM((_K,), jnp.float32),
            pltpu.VMEM((_K,), jnp.int32),
            pltpu.VMEM((_K, H), jnp.float32),
            pltpu.VMEM((8, H), jnp.float32),
            pltpu.VMEM_SHARED((_FR, H), jnp.float32),
            pltpu.SemaphoreType.DMA,
        ],
    )
    out_sc = sc(f, flat(xi), flat(cpi), flat(nci),
                flat(hydropathy), flat(mol_mass),
                flat(ang[..., 0]), flat(ang[..., 1]), flat(ang[..., 2]),
                flat(mask_aa), wsm)

    out = jnp.concatenate([out_tc, out_sc], axis=0)
    return out.reshape(B, L, H)


# speedup vs baseline: 1.5683x; 1.5683x over previous
import jax
import jax.numpy as jnp
from jax import lax
from jax.experimental import pallas as pl
from jax.experimental.pallas import tpu as pltpu
from jax.experimental.pallas import tpu_sc as plsc


_NC, _NS, _LANES = 2, 16, 16
_NW = _NC * _NS
_K = 512
_FR = 904
_SC_N = 32768
_R = 8


def _tc_body(x_ref, cp_ref, nc_ref, hyd_ref, mol_ref, a0_ref, a1_ref, a2_ref,
             m_ref, w_ref, alias_ref, out_ref):
    R, L = x_ref.shape
    w = w_ref[...]
    iota = lax.broadcasted_iota(jnp.int32, (43, L), 0)
    for r in range(R):
        xi = x_ref[r:r + 1, :]
        cpi = cp_ref[r:r + 1, :]
        nci = nc_ref[r:r + 1, :]
        oh = ((iota == xi) | (iota == cpi + 32) | (iota == nci + 39))
        m = (m_ref[r:r + 1, :] != 0).astype(jnp.float32)
        notm = 1.0 - m
        ohf = oh.astype(jnp.float32) * notm

        def angrow(a_ref):
            a = a_ref[r:r + 1, :] / 180.0
            return jnp.where(jnp.isinf(a), 0.0, a) * notm

        a_mat = jnp.concatenate(
            [ohf,
             hyd_ref[r:r + 1, :] * notm,
             mol_ref[r:r + 1, :] * notm,
             angrow(a0_ref), angrow(a1_ref), angrow(a2_ref),
             jnp.broadcast_to(m, (9, L)),
             jnp.zeros((7, L), jnp.float32)], axis=0)

        h = lax.dot_general(a_mat, w, (((0,), (0,)), ((), ())),
                            preferred_element_type=jnp.float32)
        out_ref[pl.ds(r * L, L), :] = h


def _ftable_body(w_ref, am_ref, f_ref):
    c = lax.broadcasted_iota(jnp.int32, (_FR, 48), 0)
    k = lax.broadcasted_iota(jnp.int32, (_FR, 48), 1)
    oh = ((k == c // 28) | (k == 32 + (c % 28) // 4) | (k == 39 + c % 4))
    oh = oh & (c < 896)
    h = jnp.dot(oh.astype(jnp.float32), w_ref[...],
                preferred_element_type=jnp.float32)
    mr = jnp.sum(am_ref[...], axis=0, keepdims=True)
    ci = lax.broadcasted_iota(jnp.int32, (_FR, 1), 0)
    f_ref[...] = h + jnp.where(ci == 896, 1.0, 0.0) * mr


def _build_ftable(w48, atom_mask_embed):
    H = w48.shape[1]
    return pl.pallas_call(
        _ftable_body,
        in_specs=[pl.BlockSpec((48, H), lambda: (0, 0)),
                  pl.BlockSpec((9, H), lambda: (0, 0))],
        out_specs=pl.BlockSpec((_FR, H), lambda: (0, 0)),
        out_shape=jax.ShapeDtypeStruct((_FR, H), jnp.float32),
    )(w48, atom_mask_embed)


def _sc_body(f_hbm, x_hbm, cp_hbm, nc_hbm, hyd_hbm, mol_hbm,
             a0_hbm, a1_hbm, a2_hbm, m_hbm, wsm_hbm, out_hbm,
             xv, cpv, ncv, mv, hydv, molv, a0v, a1v, a2v,
             idx_v, rows_v, wsm_v, f_sh, sem):
    H = rows_v.shape[1]
    n = x_hbm.shape[0]
    sc_base = n - _SC_N
    per_w = _SC_N // _NW
    nch = per_w // _K
    sid = lax.axis_index("s")
    wid = sid * _NC + lax.axis_index("c")
    pltpu.sync_copy(wsm_hbm, wsm_v)

    @pl.when(sid == 0)
    def _stage():
        pltpu.sync_copy(f_hbm, f_sh)
    plsc.subcore_barrier()

    @pl.loop(0, nch)
    def _chunk(ch):
        base = sc_base + wid * per_w + ch * _K
        obase = base
        pltpu.sync_copy(x_hbm.at[pl.ds(base, _K)], xv)
        pltpu.sync_copy(cp_hbm.at[pl.ds(base, _K)], cpv)
        pltpu.sync_copy(nc_hbm.at[pl.ds(base, _K)], ncv)
        pltpu.sync_copy(m_hbm.at[pl.ds(base, _K)], mv)
        pltpu.sync_copy(hyd_hbm.at[pl.ds(base, _K)], hydv)
        pltpu.sync_copy(mol_hbm.at[pl.ds(base, _K)], molv)
        pltpu.sync_copy(a0_hbm.at[pl.ds(base, _K)], a0v)
        pltpu.sync_copy(a1_hbm.at[pl.ds(base, _K)], a1v)
        pltpu.sync_copy(a2_hbm.at[pl.ds(base, _K)], a2v)

        @pl.loop(0, _K // _LANES)
        def _grp(g):
            sl = pl.ds(g * _LANES, _LANES)
            m16 = mv[sl]
            cidx = xv[sl] * 28 + cpv[sl] * 4 + ncv[sl]
            idx_v[sl] = jnp.where(m16 != 0,
                                  jnp.full((_LANES,), 896, jnp.int32), cidx)
            zf = jnp.zeros((_LANES,), jnp.float32)
            of = jnp.full((_LANES,), 1.0, jnp.float32)
            notm = jnp.where(m16 == 0, of, zf)
            hydv[sl] = hydv[sl] * notm
            molv[sl] = molv[sl] * notm
            inf = jnp.full((_LANES,), jnp.inf, jnp.float32)
            for av in (a0v, a1v, a2v):
                a = av[sl] / 180.0
                a = jnp.where(jnp.abs(a) == inf, zf, a)
                av[sl] = a * notm

        pltpu.async_copy(f_sh.at[idx_v], rows_v, sem).wait()

        @pl.loop(0, _K // _LANES)
        def _grp2(g):
            sl = pl.ds(g * _LANES, _LANES)
            svec = [hydv[sl], molv[sl], a0v[sl], a1v[sl], a2v[sl]]
            wq = [[wsm_v[j, pl.ds(q * _LANES, _LANES)] for q in range(8)]
                  for j in range(5)]
            dn = lax.GatherDimensionNumbers(
                offset_dims=(), collapsed_slice_dims=(0,),
                start_index_map=(0,))

            @pl.loop(0, _LANES)
            def _tok(ti):
                t = g * _LANES + ti
                idxv = jnp.full((_LANES,), ti, jnp.int32)
                sp = [lax.gather(s, idxv[:, None], dn, (1,),
                                 mode=lax.GatherScatterMode.PROMISE_IN_BOUNDS)
                      for s in svec]
                for q in range(8):
                    sl2 = pl.ds(q * _LANES, _LANES)
                    rows_v[t, sl2] = (rows_v[t, sl2] + sp[0] * wq[0][q] +
                                      sp[1] * wq[1][q] + sp[2] * wq[2][q] +
                                      sp[3] * wq[3][q] + sp[4] * wq[4][q])

        pltpu.sync_copy(rows_v, out_hbm.at[pl.ds(obase, _K), :])


def kernel(x, chem_polar, net_charge, hydropathy, mol_mass, ang, mask_aa,
           token_embed, atom_mask_embed, chem_polar_embed, net_charge_embed,
           hydropathy_W, mol_mass_W, angle_W):
    B, L = x.shape
    H = token_embed.shape[1]
    N = B * L
    n_tc = N - _SC_N
    b_tc = n_tc // L
    G = b_tc // _R

    w48 = jnp.concatenate(
        [token_embed, chem_polar_embed, net_charge_embed,
         jnp.zeros((5, H), jnp.float32)], axis=0)
    w64 = jnp.concatenate(
        [token_embed, chem_polar_embed, net_charge_embed,
         hydropathy_W.T, mol_mass_W.T, angle_W.T, atom_mask_embed,
         jnp.zeros((7, H), jnp.float32)], axis=0)
    wsm = jnp.concatenate(
        [hydropathy_W.T, mol_mass_W.T, angle_W.T,
         jnp.zeros((3, H), jnp.float32)], axis=0)
    f = _build_ftable(w48, atom_mask_embed)

    xi = x.astype(jnp.int32)
    cpi = chem_polar.astype(jnp.int32)
    nci = net_charge.astype(jnp.int32)
    hyd2 = hydropathy[..., 0]
    mol2 = mol_mass[..., 0]
    a02 = ang[..., 0]
    a12 = ang[..., 1]
    a22 = ang[..., 2]
    m2 = mask_aa[..., 0]

    flat = lambda v: v.reshape(N)
    sc = pl.kernel(
        _sc_body,
        out_type=jax.ShapeDtypeStruct((N, H), jnp.float32),
        mesh=plsc.VectorSubcoreMesh(core_axis_name="c", subcore_axis_name="s",
                                    num_cores=_NC, num_subcores=_NS),
        scratch_types=[
            pltpu.VMEM((_K,), jnp.int32),
            pltpu.VMEM((_K,), jnp.int32),
            pltpu.VMEM((_K,), jnp.int32),
            pltpu.VMEM((_K,), jnp.int32),
            pltpu.VMEM((_K,), jnp.float32),
            pltpu.VMEM((_K,), jnp.float32),
            pltpu.VMEM((_K,), jnp.float32),
            pltpu.VMEM((_K,), jnp.float32),
            pltpu.VMEM((_K,), jnp.float32),
            pltpu.VMEM((_K,), jnp.int32),
            pltpu.VMEM((_K, H), jnp.float32),
            pltpu.VMEM((8, H), jnp.float32),
            pltpu.VMEM_SHARED((_FR, H), jnp.float32),
            pltpu.SemaphoreType.DMA,
        ],
    )
    out_sc = sc(f, flat(xi), flat(cpi), flat(nci),
                flat(hydropathy), flat(mol_mass),
                flat(ang[..., 0]), flat(ang[..., 1]), flat(ang[..., 2]),
                flat(mask_aa), wsm)

    tok = pl.BlockSpec((_R, L), lambda i: (i, 0))
    last_blk = N // _R - 1
    out = pl.pallas_call(
        _tc_body,
        grid=(G,),
        in_specs=[tok] * 9 + [
            pl.BlockSpec((64, H), lambda i: (0, 0)),
            pl.BlockSpec((_R, H), lambda i: (last_blk, 0)),
        ],
        out_specs=pl.BlockSpec((_R * L, H), lambda i: (i, 0)),
        out_shape=jax.ShapeDtypeStruct((N, H), jnp.float32),
        input_output_aliases={10: 0},
        compiler_params=pltpu.CompilerParams(
            dimension_semantics=("arbitrary",)),
    )(xi, cpi, nci, hyd2, mol2, a02, a12, a22, m2, w64, out_sc)
    return out.reshape(B, L, H)


# speedup vs baseline: 1.6886x; 1.0767x over previous
import jax
import jax.numpy as jnp
from jax import lax
from jax.experimental import pallas as pl
from jax.experimental.pallas import tpu as pltpu
from jax.experimental.pallas import tpu_sc as plsc


_NC, _NS, _LANES = 2, 16, 16
_NW = _NC * _NS
_K = 512
_FR = 904
_SC_N = 16384
_R = 8


def _tc_body(x_ref, cp_ref, nc_ref, hyd_ref, mol_ref, a0_ref, a1_ref, a2_ref,
             m_ref, w_ref, alias_ref, out_ref):
    R, L = x_ref.shape
    w = w_ref[...]
    iota = lax.broadcasted_iota(jnp.int32, (43, L), 0)
    for r in range(R):
        xi = x_ref[r:r + 1, :]
        cpi = cp_ref[r:r + 1, :]
        nci = nc_ref[r:r + 1, :]
        oh = ((iota == xi) | (iota == cpi + 32) | (iota == nci + 39))
        m = (m_ref[r:r + 1, :] != 0).astype(jnp.float32)
        notm = 1.0 - m
        ohf = oh.astype(jnp.float32) * notm

        def angrow(a_ref):
            a = a_ref[r:r + 1, :] / 180.0
            return jnp.where(jnp.isinf(a), 0.0, a) * notm

        a_mat = jnp.concatenate(
            [ohf,
             hyd_ref[r:r + 1, :] * notm,
             mol_ref[r:r + 1, :] * notm,
             angrow(a0_ref), angrow(a1_ref), angrow(a2_ref),
             jnp.broadcast_to(m, (9, L)),
             jnp.zeros((7, L), jnp.float32)], axis=0)

        h = lax.dot_general(a_mat, w, (((0,), (0,)), ((), ())),
                            preferred_element_type=jnp.float32)
        out_ref[pl.ds(r * L, L), :] = h


def _ftable_body(w_ref, am_ref, f_ref):
    c = lax.broadcasted_iota(jnp.int32, (_FR, 48), 0)
    k = lax.broadcasted_iota(jnp.int32, (_FR, 48), 1)
    oh = ((k == c // 28) | (k == 32 + (c % 28) // 4) | (k == 39 + c % 4))
    oh = oh & (c < 896)
    h = jnp.dot(oh.astype(jnp.float32), w_ref[...],
                preferred_element_type=jnp.float32)
    mr = jnp.sum(am_ref[...], axis=0, keepdims=True)
    ci = lax.broadcasted_iota(jnp.int32, (_FR, 1), 0)
    f_ref[...] = h + jnp.where(ci == 896, 1.0, 0.0) * mr


def _build_ftable(w48, atom_mask_embed):
    H = w48.shape[1]
    return pl.pallas_call(
        _ftable_body,
        in_specs=[pl.BlockSpec((48, H), lambda: (0, 0)),
                  pl.BlockSpec((9, H), lambda: (0, 0))],
        out_specs=pl.BlockSpec((_FR, H), lambda: (0, 0)),
        out_shape=jax.ShapeDtypeStruct((_FR, H), jnp.float32),
    )(w48, atom_mask_embed)


def _sc_body(f_hbm, x_hbm, cp_hbm, nc_hbm, hyd_hbm, mol_hbm,
             a0_hbm, a1_hbm, a2_hbm, m_hbm, wsm_hbm, out_hbm,
             xv, cpv, ncv, mv, hydv, molv, a0v, a1v, a2v,
             idx_v, rows_v, wsm_v, f_sh, sem):
    H = rows_v.shape[1]
    n = x_hbm.shape[0]
    sc_base = n - _SC_N
    per_w = _SC_N // _NW
    nch = per_w // _K
    sid = lax.axis_index("s")
    wid = sid * _NC + lax.axis_index("c")
    pltpu.sync_copy(wsm_hbm, wsm_v)

    @pl.when(sid == 0)
    def _stage():
        pltpu.sync_copy(f_hbm, f_sh)
    plsc.subcore_barrier()

    @pl.loop(0, nch)
    def _chunk(ch):
        base = sc_base + wid * per_w + ch * _K
        obase = base
        pltpu.sync_copy(x_hbm.at[pl.ds(base, _K)], xv)
        pltpu.sync_copy(cp_hbm.at[pl.ds(base, _K)], cpv)
        pltpu.sync_copy(nc_hbm.at[pl.ds(base, _K)], ncv)
        pltpu.sync_copy(m_hbm.at[pl.ds(base, _K)], mv)
        pltpu.sync_copy(hyd_hbm.at[pl.ds(base, _K)], hydv)
        pltpu.sync_copy(mol_hbm.at[pl.ds(base, _K)], molv)
        pltpu.sync_copy(a0_hbm.at[pl.ds(base, _K)], a0v)
        pltpu.sync_copy(a1_hbm.at[pl.ds(base, _K)], a1v)
        pltpu.sync_copy(a2_hbm.at[pl.ds(base, _K)], a2v)

        @pl.loop(0, _K // _LANES)
        def _grp(g):
            sl = pl.ds(g * _LANES, _LANES)
            m16 = mv[sl]
            cidx = xv[sl] * 28 + cpv[sl] * 4 + ncv[sl]
            idx_v[sl] = jnp.where(m16 != 0,
                                  jnp.full((_LANES,), 896, jnp.int32), cidx)
            zf = jnp.zeros((_LANES,), jnp.float32)
            of = jnp.full((_LANES,), 1.0, jnp.float32)
            notm = jnp.where(m16 == 0, of, zf)
            hydv[sl] = hydv[sl] * notm
            molv[sl] = molv[sl] * notm
            inf = jnp.full((_LANES,), jnp.inf, jnp.float32)
            for av in (a0v, a1v, a2v):
                a = av[sl] / 180.0
                a = jnp.where(jnp.abs(a) == inf, zf, a)
                av[sl] = a * notm

        pltpu.async_copy(f_sh.at[idx_v], rows_v, sem).wait()

        @pl.loop(0, _K // _LANES)
        def _grp2(g):
            sl = pl.ds(g * _LANES, _LANES)
            svec = [hydv[sl], molv[sl], a0v[sl], a1v[sl], a2v[sl]]
            wq = [[wsm_v[j, pl.ds(q * _LANES, _LANES)] for q in range(8)]
                  for j in range(5)]
            dn = lax.GatherDimensionNumbers(
                offset_dims=(), collapsed_slice_dims=(0,),
                start_index_map=(0,))

            @pl.loop(0, _LANES)
            def _tok(ti):
                t = g * _LANES + ti
                idxv = jnp.full((_LANES,), ti, jnp.int32)
                sp = [lax.gather(s, idxv[:, None], dn, (1,),
                                 mode=lax.GatherScatterMode.PROMISE_IN_BOUNDS)
                      for s in svec]
                for q in range(8):
                    sl2 = pl.ds(q * _LANES, _LANES)
                    rows_v[t, sl2] = (rows_v[t, sl2] + sp[0] * wq[0][q] +
                                      sp[1] * wq[1][q] + sp[2] * wq[2][q] +
                                      sp[3] * wq[3][q] + sp[4] * wq[4][q])

        pltpu.sync_copy(rows_v, out_hbm.at[pl.ds(obase, _K), :])


def kernel(x, chem_polar, net_charge, hydropathy, mol_mass, ang, mask_aa,
           token_embed, atom_mask_embed, chem_polar_embed, net_charge_embed,
           hydropathy_W, mol_mass_W, angle_W):
    B, L = x.shape
    H = token_embed.shape[1]
    N = B * L
    n_tc = N - _SC_N
    b_tc = n_tc // L
    G = b_tc // _R

    w48 = jnp.concatenate(
        [token_embed, chem_polar_embed, net_charge_embed,
         jnp.zeros((5, H), jnp.float32)], axis=0)
    w64 = jnp.concatenate(
        [token_embed, chem_polar_embed, net_charge_embed,
         hydropathy_W.T, mol_mass_W.T, angle_W.T, atom_mask_embed,
         jnp.zeros((7, H), jnp.float32)], axis=0)
    wsm = jnp.concatenate(
        [hydropathy_W.T, mol_mass_W.T, angle_W.T,
         jnp.zeros((3, H), jnp.float32)], axis=0)
    f = _build_ftable(w48, atom_mask_embed)

    xi = x.astype(jnp.int32)
    cpi = chem_polar.astype(jnp.int32)
    nci = net_charge.astype(jnp.int32)
    hyd2 = hydropathy[..., 0]
    mol2 = mol_mass[..., 0]
    a02 = ang[..., 0]
    a12 = ang[..., 1]
    a22 = ang[..., 2]
    m2 = mask_aa[..., 0]

    flat = lambda v: v.reshape(N)
    sc = pl.kernel(
        _sc_body,
        out_type=jax.ShapeDtypeStruct((N, H), jnp.float32),
        mesh=plsc.VectorSubcoreMesh(core_axis_name="c", subcore_axis_name="s",
                                    num_cores=_NC, num_subcores=_NS),
        scratch_types=[
            pltpu.VMEM((_K,), jnp.int32),
            pltpu.VMEM((_K,), jnp.int32),
            pltpu.VMEM((_K,), jnp.int32),
            pltpu.VMEM((_K,), jnp.int32),
            pltpu.VMEM((_K,), jnp.float32),
            pltpu.VMEM((_K,), jnp.float32),
            pltpu.VMEM((_K,), jnp.float32),
            pltpu.VMEM((_K,), jnp.float32),
            pltpu.VMEM((_K,), jnp.float32),
            pltpu.VMEM((_K,), jnp.int32),
            pltpu.VMEM((_K, H), jnp.float32),
            pltpu.VMEM((8, H), jnp.float32),
            pltpu.VMEM_SHARED((_FR, H), jnp.float32),
            pltpu.SemaphoreType.DMA,
        ],
    )
    out_sc = sc(f, flat(xi), flat(cpi), flat(nci),
                flat(hydropathy), flat(mol_mass),
                flat(ang[..., 0]), flat(ang[..., 1]), flat(ang[..., 2]),
                flat(mask_aa), wsm)

    tok = pl.BlockSpec((_R, L), lambda i: (i, 0))
    last_blk = N // _R - 1
    out = pl.pallas_call(
        _tc_body,
        grid=(G,),
        in_specs=[tok] * 9 + [
            pl.BlockSpec((64, H), lambda i: (0, 0)),
            pl.BlockSpec((_R, H), lambda i: (last_blk, 0)),
        ],
        out_specs=pl.BlockSpec((_R * L, H), lambda i: (i, 0)),
        out_shape=jax.ShapeDtypeStruct((N, H), jnp.float32),
        input_output_aliases={10: 0},
        compiler_params=pltpu.CompilerParams(
            dimension_semantics=("arbitrary",)),
    )(xi, cpi, nci, hyd2, mol2, a02, a12, a22, m2, w64, out_sc)
    return out.reshape(B, L, H)
